# baseline (device time: 2129606 ns/iter reference)
import jax
import jax.numpy as jnp
from jax import lax
from jax.experimental import pallas as pl
from jax.experimental.pallas import tpu as pltpu


def kernel(x):
    m, n = x.shape
    half_n = n // 2

    n_chunks = 16
    n_local = 8
    rows = m // n_chunks
    lrows = m // n_local

    def body(x_ref, out_ref, local_sems, send_sems, recv_sems):
        my_x = lax.axis_index("x")
        my_y = lax.axis_index("y")
        my_z = lax.axis_index("z")
        other_y = 1 - my_y
        partner = (my_x, other_y, my_z)

        barrier_sem = pltpu.get_barrier_semaphore()
        pl.semaphore_signal(
            barrier_sem, inc=1,
            device_id=partner, device_id_type=pl.DeviceIdType.MESH,
        )
        pl.semaphore_wait(barrier_sem, 1)

        rdmas = []
        for k in range(n_chunks):
            rdma = pltpu.make_async_remote_copy(
                src_ref=x_ref.at[
                    pl.ds(k * rows, rows), pl.ds(other_y * half_n, half_n)
                ],
                dst_ref=out_ref.at[pl.ds(my_y * m + k * rows, rows), :],
                send_sem=send_sems.at[k],
                recv_sem=recv_sems.at[k],
                device_id=partner,
                device_id_type=pl.DeviceIdType.MESH,
            )
            rdma.start()
            rdmas.append(rdma)

        locals_ = []
        for k in range(n_local):
            local = pltpu.make_async_copy(
                x_ref.at[
                    pl.ds(k * lrows, lrows), pl.ds(my_y * half_n, half_n)
                ],
                out_ref.at[pl.ds(my_y * m + k * lrows, lrows), :],
                local_sems.at[k],
            )
            local.start()
            locals_.append(local)

        for local in locals_:
            local.wait()
        for rdma in rdmas:
            rdma.wait()

    return pl.pallas_call(
        body,
        out_shape=jax.ShapeDtypeStruct((2 * m, half_n), x.dtype),
        in_specs=[pl.BlockSpec(memory_space=pl.ANY)],
        out_specs=pl.BlockSpec(memory_space=pl.ANY),
        scratch_shapes=[
            pltpu.SemaphoreType.DMA((n_local,)),
            pltpu.SemaphoreType.DMA((n_chunks,)),
            pltpu.SemaphoreType.DMA((n_chunks,)),
        ],
        compiler_params=pltpu.CompilerParams(collective_id=0),
    )(x)


# device time: 811049 ns/iter; 2.6257x vs baseline; 2.6257x over previous
import jax
import jax.numpy as jnp
from jax import lax
from jax.experimental import pallas as pl
from jax.experimental.pallas import tpu as pltpu


def kernel(x):
    m, n = x.shape
    half_n = n // 2
    n_local = 16
    lrows = m // n_local

    def body(x_ref, out_ref, buf_ref, in_sems, out_sems, send_sem, recv_sem):
        my_x = lax.axis_index("x")
        my_y = lax.axis_index("y")
        my_z = lax.axis_index("z")
        other_y = 1 - my_y
        partner = (my_x, other_y, my_z)

        barrier_sem = pltpu.get_barrier_semaphore()
        pl.semaphore_signal(
            barrier_sem, inc=1,
            device_id=partner, device_id_type=pl.DeviceIdType.MESH,
        )
        pl.semaphore_wait(barrier_sem, 1)

        rdma = pltpu.make_async_remote_copy(
            src_ref=x_ref.at[:, pl.ds(other_y * half_n, half_n)],
            dst_ref=out_ref.at[pl.ds(my_y * m, m), :],
            send_sem=send_sem,
            recv_sem=recv_sem,
            device_id=partner,
            device_id_type=pl.DeviceIdType.MESH,
        )
        rdma.start()

        outs = []
        for k in range(n_local):
            slot = k % 2
            if k >= 2:
                outs[k - 2].wait()
            cp_in = pltpu.make_async_copy(
                x_ref.at[pl.ds(k * lrows, lrows), pl.ds(my_y * half_n, half_n)],
                buf_ref.at[slot],
                in_sems.at[slot],
            )
            cp_in.start()
            cp_in.wait()
            cp_out = pltpu.make_async_copy(
                buf_ref.at[slot],
                out_ref.at[pl.ds(my_y * m + k * lrows, lrows), :],
                out_sems.at[slot],
            )
            cp_out.start()
            outs.append(cp_out)
        outs[n_local - 2].wait()
        outs[n_local - 1].wait()

        rdma.wait()

    return pl.pallas_call(
        body,
        out_shape=jax.ShapeDtypeStruct((2 * m, half_n), x.dtype),
        in_specs=[pl.BlockSpec(memory_space=pl.ANY)],
        out_specs=pl.BlockSpec(memory_space=pl.ANY),
        scratch_shapes=[
            pltpu.VMEM((2, lrows, half_n), x.dtype),
            pltpu.SemaphoreType.DMA((2,)),
            pltpu.SemaphoreType.DMA((2,)),
            pltpu.SemaphoreType.DMA,
            pltpu.SemaphoreType.DMA,
        ],
        compiler_params=pltpu.CompilerParams(collective_id=0),
    )(x)
